# baseline (speedup 1.0000x reference)
import functools

import jax
import jax.numpy as jnp
from jax import lax
from jax.experimental import pallas as pl
from jax.experimental.pallas import tpu as pltpu
from jax.experimental.pallas import tpu_sc as plsc

IN_FEATURES = 4096
OUT_FEATURES = 4096
GROUPS = 32
PACK = 8
MAXQ = 15
L = 16
NC = 2
NS = 16
NW = NC * NS
RPW = IN_FEATURES // NW
PPW = RPW // PACK
PZ = OUT_FEATURES // PACK
SLAB = PACK * OUT_FEATURES
HALF = SLAB // 2
KH = HALF // PACK // L


@functools.partial(
    pl.kernel,
    mesh=plsc.VectorSubcoreMesh(core_axis_name="c", subcore_axis_name="s"),
    out_type=jax.ShapeDtypeStruct((IN_FEATURES * OUT_FEATURES,), jnp.float32),
    scratch_types=[
        pltpu.VMEM((RPW + 2 * L,), jnp.int32),
        pltpu.VMEM((OUT_FEATURES,), jnp.float32),
        pltpu.VMEM((OUT_FEATURES,), jnp.float32),
        pltpu.VMEM((PZ,), jnp.int32),
        pltpu.VMEM((HALF,), jnp.int32),
        pltpu.VMEM((HALF,), jnp.int32),
        pltpu.VMEM((HALF,), jnp.float32),
        pltpu.VMEM((HALF,), jnp.float32),
        pltpu.SemaphoreType.DMA,
        pltpu.SemaphoreType.DMA,
        pltpu.SemaphoreType.DMA,
        pltpu.SemaphoreType.DMA,
    ],
)
def _dequant(qw_hbm, qz_hbm, sc_hbm, g_hbm, out_hbm,
             g_v, sc_v, z_v, qz_v, qw0, qw1, ov0, ov1, si0, si1, so0, so1):
    wid = lax.axis_index("s") * NC + lax.axis_index("c")
    row0 = wid * RPW
    iota = lax.iota(jnp.int32, L)
    zshift = 4 * (iota & 7)

    pltpu.sync_copy(g_hbm.at[pl.ds(row0, RPW)], g_v.at[pl.ds(0, RPW)])

    def refill(gr):
        pltpu.sync_copy(sc_hbm.at[pl.ds(gr * OUT_FEATURES, OUT_FEATURES)], sc_v)
        pltpu.sync_copy(qz_hbm.at[pl.ds(gr * PZ, PZ)], qz_v)

        def zm(m, c):
            win = qz_v[pl.ds(m * L, L)]
            for t in range(PACK):
                w0 = win[2 * t]
                w1 = win[2 * t + 1]
                qzv = jnp.where(iota < PACK, w0, w1)
                z = ((qzv >> zshift) & MAXQ).astype(jnp.float32)
                z_v[pl.ds((m * PACK + t) * L, L)] = z
            return c

        lax.fori_loop(0, PZ // L, zm, 0)
        return gr

    def maybe_refill(gr, gc):
        return lax.cond(gr != gc, lambda: refill(gr), lambda: gc)

    def tile_base(k):
        return 896 * (k // 8) + 16 * k

    qwb, ovb = (qw0, qw1), (ov0, ov1)
    sib, sob = (si0, si1), (so0, so1)

    def qw_src(u):
        return qw_hbm.at[pl.ds((2 * wid + u // 2) * SLAB + (u % 2) * HALF, HALF)]

    def out_dst(u, jj):
        s = PACK * (2 * wid + u // 2) + jj
        return out_hbm.at[pl.ds(s * SLAB + (u % 2) * HALF, HALF)]

    def emit_rows(jj, j0, nj, coff, qw_v, out_v):
        def fj(j, c):
            def fk(k, c2):
                qv = qw_v[pl.ds(tile_base(k) + 128 * jj, L)]
                shift = 4 * j
                wf = ((qv >> shift) & MAXQ).astype(jnp.float32)
                sv = sc_v[pl.ds(coff + k * L, L)]
                zv = z_v[pl.ds(coff + k * L, L)]
                out_v[pl.ds(tile_base(k) + 128 * j, L)] = sv * (wf - zv)
                return c2

            lax.fori_loop(0, KH, fk, 0)
            return c

        lax.fori_loop(j0, j0 + nj, fj, 0)

    def fast_block(jj, coff, qw_v, out_v):
        def fk(k, c):
            qv = qw_v[pl.ds(tile_base(k) + 128 * jj, L)]
            sv = sc_v[pl.ds(coff + k * L, L)]
            zv = z_v[pl.ds(coff + k * L, L)]
            base = tile_base(k)
            for j in range(PACK):
                wf = ((qv >> (4 * j)) & MAXQ).astype(jnp.float32)
                out_v[pl.ds(base + 128 * j, L)] = sv * (wf - zv)
            return c

        lax.fori_loop(0, KH, fk, 0)

    pltpu.async_copy(qw_src(0), qw0, si0)
    pltpu.async_copy(qw_src(1), qw1, si1)

    gc = jnp.int32(-1)
    for ub in range(4):
        b = ub % 2
        coff = b * (OUT_FEATURES // 2)
        pltpu.make_async_copy(qw_src(0), qwb[b], sib[b]).wait()

        def sub_blocks(jp, gc, ub=ub, b=b, coff=coff):
            for jb in range(2):
                jj = 2 * jp + jb

                if ub == 0:
                    @pl.when(jp > 0)
                    def _wait_out():
                        pltpu.make_async_copy(ovb[jb], out_dst(0, 0),
                                              sob[jb]).wait()
                else:
                    pltpu.make_async_copy(ovb[jb], out_dst(0, 0), sob[jb]).wait()

                lr0 = (ub // 2) * (PACK * PACK) + jj * PACK
                gwin = g_v[pl.ds(lr0, L)]
                g0 = gwin[0]
                g7 = gwin[PACK - 1]

                def uniform(gc, jb=jb, jj=jj, g0=g0):
                    gc = maybe_refill(g0, gc)
                    fast_block(jj, coff, qwb[b], ovb[jb])
                    return gc

                def mixed(gc, jb=jb, jj=jj, gwin=gwin):
                    def fj(j, gc):
                        gj = g_v[pl.ds(lr0 + j, L)][0]
                        gc = maybe_refill(gj, gc)
                        emit_rows(jj, j, 1, coff, qwb[b], ovb[jb])
                        return gc

                    return lax.fori_loop(0, PACK, fj, gc)

                gc = lax.cond(g0 == g7, uniform, mixed, gc)
                pltpu.async_copy(ovb[jb], out_dst(ub, jj), sob[jb])
            return gc

        gc = lax.fori_loop(0, PACK // 2, sub_blocks, gc)
        if ub + 2 < 4:
            pltpu.async_copy(qw_src(ub + 2), qwb[b], sib[b])

    pltpu.make_async_copy(ov0, out_dst(0, 0), so0).wait()
    pltpu.make_async_copy(ov1, out_dst(0, 0), so1).wait()


def kernel(qweight, qzeros, scales, g_idx, num_itr=1):
    g = g_idx.astype(jnp.int32) + (jnp.asarray(num_itr, jnp.int32) - 1)
    g = jnp.clip(g, 0, GROUPS - 1)
    qwt = qweight.reshape(IN_FEATURES // PACK // PACK, PACK, OUT_FEATURES // 128, 128)
    qwt = qwt.transpose(0, 2, 1, 3).reshape(-1)
    out = _dequant(qwt, qzeros.reshape(-1), scales.reshape(-1), g)
    out = out.reshape(IN_FEATURES // PACK, OUT_FEATURES // 128, PACK, 128)
    out = out.transpose(0, 2, 1, 3)
    return out.reshape(IN_FEATURES, OUT_FEATURES)


# speedup vs baseline: 1.0099x; 1.0099x over previous
import functools

import jax
import jax.numpy as jnp
from jax import lax
from jax.experimental import pallas as pl
from jax.experimental.pallas import tpu as pltpu
from jax.experimental.pallas import tpu_sc as plsc

IN_FEATURES = 4096
OUT_FEATURES = 4096
GROUPS = 32
PACK = 8
MAXQ = 15
L = 16
NC = 2
NS = 16
NW = NC * NS
RPW = IN_FEATURES // NW
PPW = RPW // PACK
PZ = OUT_FEATURES // PACK
SLAB = PACK * OUT_FEATURES
HALF = SLAB // 2
KH = HALF // PACK // L


@functools.partial(
    pl.kernel,
    mesh=plsc.VectorSubcoreMesh(core_axis_name="c", subcore_axis_name="s"),
    out_type=jax.ShapeDtypeStruct((IN_FEATURES * OUT_FEATURES,), jnp.float32),
    scratch_types=[
        pltpu.VMEM((RPW + 2 * L,), jnp.int32),
        pltpu.VMEM((OUT_FEATURES,), jnp.float32),
        pltpu.VMEM((OUT_FEATURES,), jnp.float32),
        pltpu.VMEM((PZ,), jnp.int32),
        pltpu.VMEM((HALF,), jnp.int32),
        pltpu.VMEM((HALF,), jnp.int32),
        pltpu.VMEM((HALF,), jnp.int32),
        pltpu.VMEM((HALF,), jnp.int32),
        pltpu.VMEM((HALF,), jnp.float32),
        pltpu.VMEM((HALF,), jnp.float32),
        pltpu.SemaphoreType.DMA,
        pltpu.SemaphoreType.DMA,
        pltpu.SemaphoreType.DMA,
        pltpu.SemaphoreType.DMA,
        pltpu.SemaphoreType.DMA,
        pltpu.SemaphoreType.DMA,
    ],
)
def _dequant(qw_hbm, qz_hbm, sc_hbm, g_hbm, out_hbm,
             g_v, sc_v, z_v, qz_v, qw00, qw01, qw10, qw11, ov0, ov1,
             si00, si01, si10, si11, so0, so1):
    wid = lax.axis_index("s") * NC + lax.axis_index("c")
    row0 = wid * RPW
    iota = lax.iota(jnp.int32, L)
    zshift = 4 * (iota & 7)

    qwb = ((qw00, qw01), (qw10, qw11))
    sib = ((si00, si01), (si10, si11))
    ovb, sob = (ov0, ov1), (so0, so1)

    def qw_src(slab, h):
        return qw_hbm.at[pl.ds((2 * wid + slab) * SLAB + h * HALF, HALF)]

    def out_dst(slab, jj, h):
        s = PACK * (2 * wid + slab) + jj
        return out_hbm.at[pl.ds(s * SLAB + h * HALF, HALF)]

    for slab in range(2):
        for h in range(2):
            pltpu.async_copy(qw_src(slab, h), qwb[slab][h], sib[slab][h])

    pltpu.sync_copy(g_hbm.at[pl.ds(row0, RPW)], g_v.at[pl.ds(0, RPW)])

    def refill(gr):
        pltpu.sync_copy(sc_hbm.at[pl.ds(gr * OUT_FEATURES, OUT_FEATURES)], sc_v)
        pltpu.sync_copy(qz_hbm.at[pl.ds(gr * PZ, PZ)], qz_v)

        def zm(m, c):
            win = qz_v[pl.ds(m * L, L)]
            for t in range(PACK):
                w0 = win[2 * t]
                w1 = win[2 * t + 1]
                qzv = jnp.where(iota < PACK, w0, w1)
                z = ((qzv >> zshift) & MAXQ).astype(jnp.float32)
                z_v[pl.ds((m * PACK + t) * L, L)] = z
            return c

        lax.fori_loop(0, PZ // L, zm, 0)
        return gr

    def maybe_refill(gr, gc):
        return lax.cond(gr != gc, lambda: refill(gr), lambda: gc)

    def tile_base(k):
        return 896 * (k // 8) + 16 * k

    def emit_rows(jj, j0, nj, coff, qw_v, out_v):
        def fj(j, c):
            def fk(k, c2):
                qv = qw_v[pl.ds(tile_base(k) + 128 * jj, L)]
                shift = 4 * j
                wf = ((qv >> shift) & MAXQ).astype(jnp.float32)
                sv = sc_v[pl.ds(coff + k * L, L)]
                zv = z_v[pl.ds(coff + k * L, L)]
                out_v[pl.ds(tile_base(k) + 128 * j, L)] = sv * (wf - zv)
                return c2

            lax.fori_loop(0, KH, fk, 0)
            return c

        lax.fori_loop(j0, j0 + nj, fj, 0)

    def fast_block(jj, coff, qw_v, out_v):
        def fk(k, c):
            qv = qw_v[pl.ds(tile_base(k) + 128 * jj, L)]
            sv = sc_v[pl.ds(coff + k * L, L)]
            zv = z_v[pl.ds(coff + k * L, L)]
            base = tile_base(k)
            for j in range(PACK):
                wf = ((qv >> (4 * j)) & MAXQ).astype(jnp.float32)
                out_v[pl.ds(base + 128 * j, L)] = sv * (wf - zv)
            return c

        lax.fori_loop(0, KH, fk, 0)

    gc = jnp.int32(-1)
    for slab in range(2):
        pltpu.make_async_copy(qw_src(0, 0), qwb[slab][0], sib[slab][0]).wait()
        pltpu.make_async_copy(qw_src(0, 0), qwb[slab][1], sib[slab][1]).wait()

        def jblock(jj, gc, slab=slab):
            lr0 = slab * (PACK * PACK) + jj * PACK
            gwin = g_v[pl.ds(lr0, L)]
            g0 = gwin[0]
            g7 = gwin[PACK - 1]

            for h in range(2):
                coff = h * (OUT_FEATURES // 2)

                if slab == 0:
                    @pl.when(jj > 0)
                    def _wait_out(h=h):
                        pltpu.make_async_copy(ovb[h], out_dst(0, 0, 0),
                                              sob[h]).wait()
                else:
                    pltpu.make_async_copy(ovb[h], out_dst(0, 0, 0),
                                          sob[h]).wait()

                def uniform(gc, h=h, coff=coff, jj=jj, g0=g0, slab=slab):
                    gc = maybe_refill(g0, gc)
                    fast_block(jj, coff, qwb[slab][h], ovb[h])
                    return gc

                def mixed(gc, h=h, coff=coff, jj=jj, lr0=lr0, slab=slab):
                    def fj(j, gc):
                        gj = g_v[pl.ds(lr0 + j, L)][0]
                        gc = maybe_refill(gj, gc)
                        emit_rows(jj, j, 1, coff, qwb[slab][h], ovb[h])
                        return gc

                    return lax.fori_loop(0, PACK, fj, gc)

                gc = lax.cond(g0 == g7, uniform, mixed, gc)
                pltpu.async_copy(ovb[h], out_dst(slab, jj, h), sob[h])
            return gc

        gc = lax.fori_loop(0, PACK, jblock, gc)

    pltpu.make_async_copy(ov0, out_dst(0, 0, 0), so0).wait()
    pltpu.make_async_copy(ov1, out_dst(0, 0, 0), so1).wait()


def kernel(qweight, qzeros, scales, g_idx, num_itr=1):
    g = g_idx.astype(jnp.int32) + (jnp.asarray(num_itr, jnp.int32) - 1)
    g = jnp.clip(g, 0, GROUPS - 1)
    qwt = qweight.reshape(IN_FEATURES // PACK // PACK, PACK, OUT_FEATURES // 128, 128)
    qwt = qwt.transpose(0, 2, 1, 3).reshape(-1)
    out = _dequant(qwt, qzeros.reshape(-1), scales.reshape(-1), g)
    out = out.reshape(IN_FEATURES // PACK, OUT_FEATURES // 128, PACK, 128)
    out = out.transpose(0, 2, 1, 3)
    return out.reshape(IN_FEATURES, OUT_FEATURES)
